# baseline (speedup 1.0000x reference)
import functools

import jax
import jax.numpy as jnp
from jax import lax
from jax.experimental import pallas as pl
from jax.experimental.pallas import tpu as pltpu
from jax.experimental.pallas import tpu_sc as plsc

NUM_EMBEDDINGS = 100000
EMBEDDING_DIM = 128
BATCH = 4096
SEQ = 50
N_TOKENS = BATCH * SEQ

_INFO = plsc.get_sparse_core_info()
_NW = _INFO.num_cores * _INFO.num_subcores
_PER_W = N_TOKENS // _NW
_CHUNK = 640
_NSTEP = _PER_W // _CHUNK


def _sc_gather(ids_flat, table):
  mesh = plsc.VectorSubcoreMesh(core_axis_name="c", subcore_axis_name="s")

  @functools.partial(
      pl.kernel,
      out_type=jax.ShapeDtypeStruct((N_TOKENS, EMBEDDING_DIM), jnp.float32),
      mesh=mesh,
      scratch_types=[
          pltpu.VMEM((_PER_W,), jnp.int32),
          pltpu.VMEM((_CHUNK, EMBEDDING_DIM), jnp.float32),
          pltpu.SemaphoreType.DMA,
      ],
  )
  def body(ids_hbm, table_hbm, out_hbm, idx_all, rows_v, sem):
    wid = lax.axis_index("s") * _INFO.num_cores + lax.axis_index("c")
    base = wid * _PER_W

    pltpu.sync_copy(ids_hbm.at[pl.ds(base, _PER_W)], idx_all)

    for g in range(_NSTEP):
      pltpu.async_copy(
          table_hbm.at[idx_all.at[pl.ds(g * _CHUNK, _CHUNK)]],
          rows_v, sem).wait()
      pltpu.sync_copy(rows_v, out_hbm.at[pl.ds(base + g * _CHUNK, _CHUNK)])

  return body(ids_flat, table)


def kernel(input_ids, table):
  ids_t_flat = input_ids.T.reshape(N_TOKENS)
  out = _sc_gather(ids_t_flat, table)
  return out.reshape(SEQ, BATCH, EMBEDDING_DIM).transpose(1, 0, 2)
